# Initial kernel scaffold; baseline (speedup 1.0000x reference)
import jax
import jax.numpy as jnp
from jax.experimental import pallas as pl


def kernel(x, W_in, conv_w, conv_b, A_param, dt_bias, W_out):
    raise NotImplementedError("write your pallas kernel here")


# speedup vs baseline: 3.2428x; 3.2428x over previous
import functools

import jax
import jax.numpy as jnp
from jax.experimental import pallas as pl
from jax.experimental.pallas import tpu as pltpu

CHUNK = 64
D_CONV = 4


def _proj_kernel(x_ref, w_ref, o_ref):
    xb = x_ref[...].astype(jnp.bfloat16)
    o_ref[...] = jax.lax.dot_general(
        xb, w_ref[...],
        dimension_numbers=(((1,), (0,)), ((), ())),
        preferred_element_type=jnp.float32)


def _proj_xbc(x0, w1t_bf, *, interpret=False):
    s, dm = x0.shape
    n = w1t_bf.shape[1]
    bm, bn = 512, 1024
    return pl.pallas_call(
        _proj_kernel,
        grid=(s // bm, n // bn),
        in_specs=[
            pl.BlockSpec((bm, dm), lambda i, j: (i, 0)),
            pl.BlockSpec((dm, bn), lambda i, j: (0, j)),
        ],
        out_specs=pl.BlockSpec((bm, bn), lambda i, j: (i, j)),
        out_shape=jax.ShapeDtypeStruct((s, n), jnp.float32),
        compiler_params=pltpu.CompilerParams(
            dimension_semantics=("parallel", "parallel")),
        name="proj_xbc",
        interpret=interpret,
    )(x0, w1t_bf)


def _ssd_kernel(nheads, nchunks,
                cur_ref, prev_ref, x_ref, wdt_ref, cw_ref, cb_ref,
                ap_ref, dtb_ref, y_ref, inv_ref, state_ref, carry_ref):
    i = pl.program_id(0)
    hp = 64

    @pl.when(i == 0)
    def _init():
        state_ref[...] = jnp.zeros_like(state_ref)
        carry_ref[...] = jnp.zeros_like(carry_ref)

    f32 = jnp.float32
    cur = cur_ref[...]
    tail = prev_ref[61:64, :]
    tail = jnp.where(i == 0, 0.0, tail)
    ext = jnp.concatenate([tail, cur], axis=0)
    conv = cur * cw_ref[3:4, :] + cb_ref[...]
    conv += ext[2:66, :] * cw_ref[2:3, :]
    conv += ext[1:65, :] * cw_ref[1:2, :]
    conv += ext[0:64, :] * cw_ref[0:1, :]

    nb = x_ref.shape[0]
    rows = nb * CHUNK
    xall = x_ref[...].reshape(rows, x_ref.shape[2])
    dtraw = jax.lax.dot_general(
        xall, wdt_ref[...], dimension_numbers=(((1,), (0,)), ((), ())),
        preferred_element_type=f32,
        precision=jax.lax.Precision.HIGHEST) + dtb_ref[...]
    dt = jnp.maximum(dtraw, 0.0) + jnp.log1p(jnp.exp(-jnp.abs(dtraw)))
    a_all = ap_ref[...] * dt

    ii = jax.lax.broadcasted_iota(jnp.int32, (rows, rows), 0)
    jj = jax.lax.broadcasted_iota(jnp.int32, (rows, rows), 1)
    blkmask = ((jj <= ii) & ((ii // CHUNK) == (jj // CHUNK))).astype(f32)
    cuml = jax.lax.dot_general(
        blkmask, a_all, dimension_numbers=(((1,), (0,)), ((), ())),
        preferred_element_type=f32,
        precision=jax.lax.Precision.HIGHEST)
    coff = carry_ref[0:4, :]
    rsel = ((ii[:, 0:4] // CHUNK) ==
            jax.lax.broadcasted_iota(jnp.int32, (rows, 4), 1)).astype(f32)
    cuma = cuml + jax.lax.dot_general(
        rsel, coff, dimension_numbers=(((1,), (0,)), ((), ())),
        preferred_element_type=f32,
        precision=jax.lax.Precision.HIGHEST)
    en = jnp.exp(-cuma)
    inner = jax.lax.dot_general(
        blkmask, en, dimension_numbers=(((1,), (0,)), ((), ())),
        preferred_element_type=f32,
        precision=jax.lax.Precision.HIGHEST)
    inner += jax.lax.dot_general(
        rsel, carry_ref[4:8, :], dimension_numbers=(((1,), (0,)), ((), ())),
        preferred_element_type=f32,
        precision=jax.lax.Precision.HIGHEST)
    inv_ref[...] = (1.0 / (jnp.exp(cuma) * inner)).reshape(nb, CHUNK, nheads)
    newoff = jnp.concatenate(
        [cuma[b * CHUNK + CHUNK - 1:b * CHUNK + CHUNK, :] for b in range(nb)],
        axis=0)
    segsum = jnp.concatenate(
        [jnp.sum(en[b * CHUNK:(b + 1) * CHUNK, :], axis=0, keepdims=True)
         for b in range(nb)], axis=0)
    carry_ref[0:4, :] = newoff
    carry_ref[4:8, :] = carry_ref[4:8, :] + segsum

    cuma0 = cuml[0:CHUNK, :]

    u = jnp.exp(cuma0)
    v = jnp.exp(-cuma0)
    ul = u[CHUNK - 1:CHUNK, :]
    li = jax.lax.broadcasted_iota(jnp.int32, (CHUNK, CHUNK), 0)
    lj = jax.lax.broadcasted_iota(jnp.int32, (CHUNK, CHUNK), 1)
    ltri = lj <= li
    ds = hp * nheads
    bf = jnp.bfloat16
    for h in range(nheads):
        sl = slice(h * hp, (h + 1) * hp)
        ucol = u[:, h:h + 1]
        vcol = v[:, h:h + 1]
        ct = (conv[:, sl] * ucol).astype(bf)
        bv = (conv[:, ds + h * hp:ds + (h + 1) * hp] * vcol).astype(bf)
        xh = conv[:, 2 * ds + h * hp:2 * ds + (h + 1) * hp].astype(bf)
        g = jax.lax.dot_general(
            ct, bv, dimension_numbers=(((1,), (1,)), ((), ())),
            preferred_element_type=f32)
        gm = jnp.where(ltri, g, 0.0).astype(bf)
        yd = jax.lax.dot_general(
            gm, xh, dimension_numbers=(((1,), (0,)), ((), ())),
            preferred_element_type=f32)
        sh = state_ref[sl, :]
        yo = jax.lax.dot_general(
            ct, sh.astype(bf), dimension_numbers=(((1,), (0,)), ((), ())),
            preferred_element_type=f32)
        y_ref[:, sl] = yd + yo
        sc = jax.lax.dot_general(
            bv, xh, dimension_numbers=(((0,), (0,)), ((), ())),
            preferred_element_type=f32)
        state_ref[sl, :] = (sh + sc) * ul[0:1, h:h + 1]


def _ssd(proj0, x, wdt_t, convw_t, convb2, ap_row, dtb_row, *,
         interpret=False):
    nb, s, dm = x.shape
    dcc = proj0.shape[1]
    nheads = ap_row.shape[1]
    nchunks = s // CHUNK
    kfn = functools.partial(_ssd_kernel, nheads, nchunks)
    return pl.pallas_call(
        kfn,
        grid=(nchunks,),
        in_specs=[
            pl.BlockSpec((CHUNK, dcc), lambda i: (i, 0)),
            pl.BlockSpec((CHUNK, dcc), lambda i: (jnp.maximum(i - 1, 0), 0)),
            pl.BlockSpec((nb, CHUNK, dm), lambda i: (0, i, 0)),
            pl.BlockSpec((dm, nheads), lambda i: (0, 0)),
            pl.BlockSpec((D_CONV, dcc), lambda i: (0, 0)),
            pl.BlockSpec((1, dcc), lambda i: (0, 0)),
            pl.BlockSpec((1, nheads), lambda i: (0, 0)),
            pl.BlockSpec((1, nheads), lambda i: (0, 0)),
        ],
        out_specs=[
            pl.BlockSpec((CHUNK, dcc // 3), lambda i: (i, 0)),
            pl.BlockSpec((nb, CHUNK, nheads), lambda i: (0, i, 0)),
        ],
        out_shape=[
            jax.ShapeDtypeStruct((s, dcc // 3), jnp.float32),
            jax.ShapeDtypeStruct((nb, s, nheads), jnp.float32),
        ],
        scratch_shapes=[
            pltpu.VMEM((dcc // 3, CHUNK), jnp.float32),
            pltpu.VMEM((8, nheads), jnp.float32),
        ],
        compiler_params=pltpu.CompilerParams(
            dimension_semantics=("arbitrary",)),
        name="conv_ssd_norm",
        interpret=interpret,
    )(proj0, proj0, x, wdt_t, convw_t, convb2, ap_row, dtb_row)


def _out_kernel(nheads, y_ref, inv_ref, w_ref, o_ref):
    f32 = jnp.float32
    bm = y_ref.shape[0]
    di = y_ref.shape[1]
    hp = di // nheads
    inv = inv_ref[...].reshape(bm, nheads)
    hh = jax.lax.broadcasted_iota(jnp.int32, (nheads, di), 0)
    cc = jax.lax.broadcasted_iota(jnp.int32, (nheads, di), 1)
    esel = ((cc // hp) == hh).astype(f32)
    invx = jax.lax.dot_general(
        inv, esel, dimension_numbers=(((1,), (0,)), ((), ())),
        preferred_element_type=f32,
        precision=jax.lax.Precision.HIGHEST)
    z = (y_ref[...] * invx).astype(jnp.bfloat16)
    o = jax.lax.dot_general(
        z, w_ref[...], dimension_numbers=(((1,), (0,)), ((), ())),
        preferred_element_type=f32)
    o_ref[...] = o.reshape(1, bm, o.shape[1])


def _out_proj(y0, invn, wot_bf, *, interpret=False):
    nb, s, nheads = invn.shape
    di = y0.shape[1]
    dm = wot_bf.shape[1]
    bm = 512
    kfn = functools.partial(_out_kernel, nheads)
    return pl.pallas_call(
        kfn,
        grid=(nb, s // bm),
        in_specs=[
            pl.BlockSpec((bm, di), lambda b, m: (m, 0)),
            pl.BlockSpec((1, bm, nheads), lambda b, m: (b, m, 0)),
            pl.BlockSpec((di, dm), lambda b, m: (0, 0)),
        ],
        out_specs=pl.BlockSpec((1, bm, dm), lambda b, m: (b, m, 0)),
        out_shape=jax.ShapeDtypeStruct((nb, s, dm), jnp.float32),
        compiler_params=pltpu.CompilerParams(
            dimension_semantics=("parallel", "parallel")),
        name="scale_outproj",
        interpret=interpret,
    )(y0, invn, wot_bf)


def kernel(x, W_in, conv_w, conv_b, A_param, dt_bias, W_out,
           interpret=False):
    nb, s, dm = x.shape
    nheads = A_param.shape[0]
    dcc = conv_w.shape[0]

    x0 = x[0]
    w1t_bf = W_in[:dcc].T.astype(jnp.bfloat16)
    wdt_t = W_in[dcc:].T
    convw_t = conv_w.T
    convb2 = conv_b.reshape(1, dcc)
    ap_row = A_param.reshape(1, nheads)
    dtb_row = dt_bias.reshape(1, nheads)
    wot_bf = W_out.T.astype(jnp.bfloat16)

    proj0 = _proj_xbc(x0, w1t_bf, interpret=interpret)
    y0, invn = _ssd(proj0, x, wdt_t, convw_t, convb2, ap_row, dtb_row,
                    interpret=interpret)
    return _out_proj(y0, invn, wot_bf, interpret=interpret)


# speedup vs baseline: 4.1560x; 1.2816x over previous
import functools

import jax
import jax.numpy as jnp
from jax.experimental import pallas as pl
from jax.experimental.pallas import tpu as pltpu

CHUNK = 64
D_CONV = 4
NCORES = 1
HP = 64
HIGH = jax.lax.Precision.HIGHEST


def _proj_kernel(x_ref, w_ref, o_ref):
    xb = x_ref[...].astype(jnp.bfloat16)
    o_ref[...] = jax.lax.dot_general(
        xb, w_ref[...],
        dimension_numbers=(((1,), (0,)), ((), ())),
        preferred_element_type=jnp.float32)


def _proj_xbc(x0, w1t_bf):
    s, dm = x0.shape
    n = w1t_bf.shape[1]
    bm, bn = 512, 1024
    mh = s // bm // NCORES
    return pl.pallas_call(
        _proj_kernel,
        grid=(NCORES, mh, n // bn),
        in_specs=[
            pl.BlockSpec((bm, dm), lambda c, i, j: (c * mh + i, 0)),
            pl.BlockSpec((dm, bn), lambda c, i, j: (0, j)),
        ],
        out_specs=pl.BlockSpec((bm, bn), lambda c, i, j: (c * mh + i, j)),
        out_shape=jax.ShapeDtypeStruct((s, n), jnp.float32),
        compiler_params=pltpu.CompilerParams(
            dimension_semantics=("core_parallel", "parallel", "parallel")),
        name="proj_xbc",
    )(x0, w1t_bf)


def _ssd_kernel(nheads,
                cur_ref, prev_ref, xb2_ref, x0c_ref, wdt_ref, wdtp_ref,
                cw_ref, cb_ref, ap_ref, dtb_ref, app_ref, dtbp_ref,
                y_ref, inv_ref, state_ref, carry_ref):
    i = pl.program_id(1)
    f32 = jnp.float32
    bf = jnp.bfloat16
    nh_loc = nheads // NCORES
    part = nh_loc * HP

    @pl.when(i == 0)
    def _init():
        state_ref[...] = jnp.zeros_like(state_ref)
        carry_ref[...] = jnp.zeros_like(carry_ref)

    cur = cur_ref[...]
    tail = prev_ref[CHUNK - 3:CHUNK, :]
    tail = jnp.where(i == 0, 0.0, tail)
    ext = jnp.concatenate([tail, cur], axis=0)
    conv = cur * cw_ref[3:4, :] + cb_ref[...]
    conv += ext[2:CHUNK + 2, :] * cw_ref[2:3, :]
    conv += ext[1:CHUNK + 1, :] * cw_ref[1:2, :]
    conv += ext[0:CHUNK, :] * cw_ref[0:1, :]

    nbl = xb2_ref.shape[0]
    rows = nbl * CHUNK
    xall = xb2_ref[...].reshape(rows, xb2_ref.shape[2]).astype(bf)
    dtraw = jax.lax.dot_general(
        xall, wdt_ref[...], dimension_numbers=(((1,), (0,)), ((), ())),
        preferred_element_type=f32) + dtb_ref[...]
    dt = jnp.maximum(dtraw, 0.0) + jnp.log1p(jnp.exp(-jnp.abs(dtraw)))
    a_all = ap_ref[...] * dt

    ii = jax.lax.broadcasted_iota(jnp.int32, (rows, rows), 0)
    jj = jax.lax.broadcasted_iota(jnp.int32, (rows, rows), 1)
    blkmask = ((jj <= ii) & ((ii // CHUNK) == (jj // CHUNK))).astype(f32)
    cuml = jax.lax.dot_general(
        blkmask, a_all, dimension_numbers=(((1,), (0,)), ((), ())),
        preferred_element_type=f32, precision=HIGH)
    coff = carry_ref[0:nbl, :]
    rsel = ((ii[:, 0:nbl] // CHUNK) ==
            jax.lax.broadcasted_iota(jnp.int32, (rows, nbl), 1)).astype(f32)
    cuma = cuml + jax.lax.dot_general(
        rsel, coff, dimension_numbers=(((1,), (0,)), ((), ())),
        preferred_element_type=f32, precision=HIGH)
    en = jnp.exp(-cuma)
    inner = jax.lax.dot_general(
        blkmask.astype(bf), en.astype(bf),
        dimension_numbers=(((1,), (0,)), ((), ())),
        preferred_element_type=f32)
    inner += jax.lax.dot_general(
        rsel, carry_ref[4:4 + nbl, :],
        dimension_numbers=(((1,), (0,)), ((), ())),
        preferred_element_type=f32, precision=HIGH)
    inv_ref[...] = (1.0 / (jnp.exp(cuma) * inner)).reshape(nbl, CHUNK, nheads)
    newoff = jnp.concatenate(
        [cuma[b * CHUNK + CHUNK - 1:b * CHUNK + CHUNK, :] for b in range(nbl)],
        axis=0)
    segsum = jnp.concatenate(
        [jnp.sum(en[b * CHUNK:(b + 1) * CHUNK, :], axis=0, keepdims=True)
         for b in range(nbl)], axis=0)
    carry_ref[0:nbl, :] = newoff
    carry_ref[4:4 + nbl, :] = carry_ref[4:4 + nbl, :] + segsum

    x0 = x0c_ref[...].reshape(CHUNK, x0c_ref.shape[2]).astype(bf)
    dtraw0 = jax.lax.dot_general(
        x0, wdtp_ref[...], dimension_numbers=(((1,), (0,)), ((), ())),
        preferred_element_type=f32) + dtbp_ref[...]
    dt0 = jnp.maximum(dtraw0, 0.0) + jnp.log1p(jnp.exp(-jnp.abs(dtraw0)))
    a0 = (app_ref[...] * dt0).astype(bf)
    li = jax.lax.broadcasted_iota(jnp.int32, (CHUNK, CHUNK), 0)
    lj = jax.lax.broadcasted_iota(jnp.int32, (CHUNK, CHUNK), 1)
    ltri = lj <= li
    cum0 = jax.lax.dot_general(
        ltri.astype(bf), a0, dimension_numbers=(((1,), (0,)), ((), ())),
        preferred_element_type=f32)
    u = jnp.exp(cum0)
    v = jnp.exp(-cum0)
    ul = u[CHUNK - 1:CHUNK, :]
    for h in range(nh_loc):
        sl = slice(h * HP, (h + 1) * HP)
        ucol = u[:, h:h + 1]
        vcol = v[:, h:h + 1]
        ct = (conv[:, sl] * ucol).astype(bf)
        bv = (conv[:, part + h * HP:part + (h + 1) * HP] * vcol).astype(bf)
        xh = conv[:, 2 * part + h * HP:2 * part + (h + 1) * HP].astype(bf)
        g = jax.lax.dot_general(
            ct, bv, dimension_numbers=(((1,), (1,)), ((), ())),
            preferred_element_type=f32)
        gm = jnp.where(ltri, g, 0.0).astype(bf)
        yd = jax.lax.dot_general(
            gm, xh, dimension_numbers=(((1,), (0,)), ((), ())),
            preferred_element_type=f32)
        sh = state_ref[sl, :]
        yo = jax.lax.dot_general(
            ct, sh.astype(bf), dimension_numbers=(((1,), (0,)), ((), ())),
            preferred_element_type=f32)
        y_ref[:, sl] = yd + yo
        sc = jax.lax.dot_general(
            bv, xh, dimension_numbers=(((0,), (0,)), ((), ())),
            preferred_element_type=f32)
        state_ref[sl, :] = (sh + sc) * ul[0:1, h:h + 1]


def _ssd(proj0p, x, wdt_bf, wdtp_bf, cwp, cbp, ap_row, dtb_row,
         app_row, dtbp_row):
    nb, s, dm = x.shape
    dcc = proj0p.shape[1]
    nheads = ap_row.shape[1]
    nchunks = s // CHUNK
    slab = dcc // NCORES
    nbl = nb // NCORES
    kfn = functools.partial(_ssd_kernel, nheads)
    return pl.pallas_call(
        kfn,
        grid=(NCORES, nchunks),
        in_specs=[
            pl.BlockSpec((CHUNK, slab), lambda c, i: (i, c)),
            pl.BlockSpec((CHUNK, slab),
                         lambda c, i: (jnp.maximum(i - 1, 0), c)),
            pl.BlockSpec((nbl, CHUNK, dm), lambda c, i: (c, i, 0)),
            pl.BlockSpec((1, CHUNK, dm), lambda c, i: (0, i, 0)),
            pl.BlockSpec((dm, nheads), lambda c, i: (0, 0)),
            pl.BlockSpec((dm, 128), lambda c, i: (0, c)),
            pl.BlockSpec((D_CONV, slab), lambda c, i: (0, c)),
            pl.BlockSpec((1, slab), lambda c, i: (0, c)),
            pl.BlockSpec((1, nheads), lambda c, i: (0, 0)),
            pl.BlockSpec((1, nheads), lambda c, i: (0, 0)),
            pl.BlockSpec((1, 128), lambda c, i: (0, c)),
            pl.BlockSpec((1, 128), lambda c, i: (0, c)),
        ],
        out_specs=[
            pl.BlockSpec((CHUNK, slab // 3), lambda c, i: (i, c)),
            pl.BlockSpec((nbl, CHUNK, nheads), lambda c, i: (c, i, 0)),
        ],
        out_shape=[
            jax.ShapeDtypeStruct((s, dcc // 3), jnp.float32),
            jax.ShapeDtypeStruct((nb, s, nheads), jnp.float32),
        ],
        scratch_shapes=[
            pltpu.VMEM((slab // 3, HP), jnp.float32),
            pltpu.VMEM((8, nheads), jnp.float32),
        ],
        compiler_params=pltpu.CompilerParams(
            dimension_semantics=("core_parallel", "arbitrary")),
        name="conv_ssd_norm",
    )(proj0p, proj0p, x, x, wdt_bf, wdtp_bf, cwp, cbp, ap_row, dtb_row,
      app_row, dtbp_row)


def _out_kernel(nheads, y_ref, inv_ref, w_ref, o_ref):
    f32 = jnp.float32
    bm = y_ref.shape[0]
    di = y_ref.shape[1]
    hp = di // nheads
    inv = inv_ref[...].reshape(bm, nheads)
    hh = jax.lax.broadcasted_iota(jnp.int32, (nheads, di), 0)
    cc = jax.lax.broadcasted_iota(jnp.int32, (nheads, di), 1)
    esel = ((cc // hp) == hh).astype(f32)
    invx = jax.lax.dot_general(
        inv, esel, dimension_numbers=(((1,), (0,)), ((), ())),
        preferred_element_type=f32, precision=HIGH)
    z = (y_ref[...] * invx).astype(jnp.bfloat16)
    o = jax.lax.dot_general(
        z, w_ref[...], dimension_numbers=(((1,), (0,)), ((), ())),
        preferred_element_type=f32)
    o_ref[...] = o.reshape(1, bm, o.shape[1])


def _out_proj(y0, invn, wot_bf):
    nb, s, nheads = invn.shape
    di = y0.shape[1]
    dm = wot_bf.shape[1]
    bm = 512
    nbl = nb // NCORES
    kfn = functools.partial(_out_kernel, nheads)
    return pl.pallas_call(
        kfn,
        grid=(NCORES, nbl, s // bm),
        in_specs=[
            pl.BlockSpec((bm, di), lambda c, b, m: (m, 0)),
            pl.BlockSpec((1, bm, nheads),
                         lambda c, b, m: (c * nbl + b, m, 0)),
            pl.BlockSpec((di, dm), lambda c, b, m: (0, 0)),
        ],
        out_specs=pl.BlockSpec((1, bm, dm),
                               lambda c, b, m: (c * nbl + b, m, 0)),
        out_shape=jax.ShapeDtypeStruct((nb, s, dm), jnp.float32),
        compiler_params=pltpu.CompilerParams(
            dimension_semantics=("core_parallel", "parallel", "parallel")),
        name="scale_outproj",
    )(y0, invn, wot_bf)


def kernel(x, W_in, conv_w, conv_b, A_param, dt_bias, W_out):
    nb, s, dm = x.shape
    nheads = A_param.shape[0]
    dcc = conv_w.shape[0]
    nh_loc = nheads // NCORES

    def permute_cols(a):
        lead = a.shape[:-1]
        ap = a.reshape(*lead, 3, NCORES, nh_loc, HP)
        ap = jnp.moveaxis(ap, -4, -3)
        return ap.reshape(*lead, dcc)

    x0 = x[0]
    w1t_bf = permute_cols(W_in[:dcc].T).astype(jnp.bfloat16)
    wdt = W_in[dcc:].T
    wdt_bf = wdt.astype(jnp.bfloat16)
    wdtp = jnp.zeros((dm, NCORES * 128), jnp.float32)
    app_row = jnp.zeros((1, NCORES * 128), jnp.float32)
    dtbp_row = jnp.zeros((1, NCORES * 128), jnp.float32)
    for c in range(NCORES):
        hs = slice(c * nh_loc, (c + 1) * nh_loc)
        cs = slice(c * 128, c * 128 + nh_loc)
        wdtp = wdtp.at[:, cs].set(wdt[:, hs])
        app_row = app_row.at[0, cs].set(A_param[hs])
        dtbp_row = dtbp_row.at[0, cs].set(dt_bias[hs])
    wdtp_bf = wdtp.astype(jnp.bfloat16)
    cwp = permute_cols(conv_w.T)
    cbp = permute_cols(conv_b.reshape(1, dcc))
    ap_row = A_param.reshape(1, nheads)
    dtb_row = dt_bias.reshape(1, nheads)
    wot_bf = W_out.T.astype(jnp.bfloat16)

    proj0p = _proj_xbc(x0, w1t_bf)
    y0p, invn = _ssd(proj0p, x, wdt_bf, wdtp_bf, cwp, cbp, ap_row, dtb_row,
                     app_row, dtbp_row)
    return _out_proj(y0p, invn, wot_bf)


# speedup vs baseline: 4.2716x; 1.0278x over previous
import functools

import jax
import jax.numpy as jnp
from jax.experimental import pallas as pl
from jax.experimental.pallas import tpu as pltpu

CHUNK = 64
D_CONV = 4
NCORES = 1
HP = 64
HIGH = jax.lax.Precision.HIGHEST


def _proj_kernel(x_ref, w_ref, o_ref):
    xb = x_ref[...].astype(jnp.bfloat16)
    o_ref[...] = jax.lax.dot_general(
        xb, w_ref[...],
        dimension_numbers=(((1,), (0,)), ((), ())),
        preferred_element_type=jnp.float32)


def _proj_xbc(x0, w1t_bf):
    s, dm = x0.shape
    n = w1t_bf.shape[1]
    bm, bn = 512, 1024
    mh = s // bm // NCORES
    return pl.pallas_call(
        _proj_kernel,
        grid=(NCORES, mh, n // bn),
        in_specs=[
            pl.BlockSpec((bm, dm), lambda c, i, j: (c * mh + i, 0)),
            pl.BlockSpec((dm, bn), lambda c, i, j: (0, j)),
        ],
        out_specs=pl.BlockSpec((bm, bn), lambda c, i, j: (c * mh + i, j)),
        out_shape=jax.ShapeDtypeStruct((s, n), jnp.float32),
        compiler_params=pltpu.CompilerParams(
            dimension_semantics=("core_parallel", "parallel", "parallel")),
        name="proj_xbc",
    )(x0, w1t_bf)


def _ssd_kernel(nheads,
                cur_ref, prev_ref, xb2_ref, x0c_ref, wdt_ref, wdtp_ref,
                cw_ref, cb_ref, ap_ref, dtb_ref, app_ref, dtbp_ref,
                y_ref, inv_ref, state_ref, carry_ref):
    i = pl.program_id(1)
    f32 = jnp.float32
    bf = jnp.bfloat16
    nh_loc = nheads // NCORES
    part = nh_loc * HP

    @pl.when(i == 0)
    def _init():
        state_ref[...] = jnp.zeros_like(state_ref)
        carry_ref[...] = jnp.zeros_like(carry_ref)

    cur = cur_ref[...]
    tail8 = jnp.where(i > 0, prev_ref[CHUNK - 8:CHUNK, :], 0.0)
    ext72 = jnp.concatenate([cur, tail8], axis=0).astype(bf)
    i72 = jax.lax.broadcasted_iota(jnp.int32, (CHUNK, CHUNK + 8), 0)
    j72 = jax.lax.broadcasted_iota(jnp.int32, (CHUNK, CHUNK + 8), 1)
    conv = cur * cw_ref[3:4, :] + cb_ref[...]
    for k in (1, 2, 3):
        mk = (((j72 == i72 - k) & (j72 < CHUNK)) |
              ((j72 == CHUNK + 8 - k + i72) & (i72 < k)))
        shifted = jax.lax.dot_general(
            mk.astype(bf), ext72, dimension_numbers=(((1,), (0,)), ((), ())),
            preferred_element_type=f32)
        conv += shifted * cw_ref[3 - k:4 - k, :]

    nbl = xb2_ref.shape[0]
    rows = nbl * CHUNK
    xall = xb2_ref[...].reshape(rows, xb2_ref.shape[2]).astype(bf)
    dtraw = jax.lax.dot_general(
        xall, wdt_ref[...], dimension_numbers=(((1,), (0,)), ((), ())),
        preferred_element_type=f32) + dtb_ref[...]
    dt = jnp.maximum(dtraw, 0.0) + jnp.log1p(jnp.exp(-jnp.abs(dtraw)))
    a_all = ap_ref[...] * dt

    ii = jax.lax.broadcasted_iota(jnp.int32, (rows, rows), 0)
    jj = jax.lax.broadcasted_iota(jnp.int32, (rows, rows), 1)
    blkmask = ((jj <= ii) & ((ii // CHUNK) == (jj // CHUNK))).astype(f32)
    ahi = a_all.astype(bf)
    alo = (a_all - ahi.astype(f32)).astype(bf)
    blk_bf = blkmask.astype(bf)
    cuml = (jax.lax.dot_general(
                blk_bf, ahi, dimension_numbers=(((1,), (0,)), ((), ())),
                preferred_element_type=f32) +
            jax.lax.dot_general(
                blk_bf, alo, dimension_numbers=(((1,), (0,)), ((), ())),
                preferred_element_type=f32))
    coff = carry_ref[0:nbl, :]
    rsel = ((ii[:, 0:nbl] // CHUNK) ==
            jax.lax.broadcasted_iota(jnp.int32, (rows, nbl), 1)).astype(f32)
    cuma = cuml + jax.lax.dot_general(
        rsel, coff, dimension_numbers=(((1,), (0,)), ((), ())),
        preferred_element_type=f32, precision=HIGH)
    en = jnp.exp(-cuma)
    inner = jax.lax.dot_general(
        blk_bf, en.astype(bf),
        dimension_numbers=(((1,), (0,)), ((), ())),
        preferred_element_type=f32)
    inner += jax.lax.dot_general(
        rsel, carry_ref[4:4 + nbl, :],
        dimension_numbers=(((1,), (0,)), ((), ())),
        preferred_element_type=f32, precision=HIGH)
    inv_ref[...] = (1.0 / (jnp.exp(cuma) * inner)).reshape(nbl, CHUNK, nheads)
    newoff = jnp.concatenate(
        [cuma[b * CHUNK + CHUNK - 1:b * CHUNK + CHUNK, :] for b in range(nbl)],
        axis=0)
    segsum = jnp.concatenate(
        [jnp.sum(en[b * CHUNK:(b + 1) * CHUNK, :], axis=0, keepdims=True)
         for b in range(nbl)], axis=0)
    carry_ref[0:nbl, :] = newoff
    carry_ref[4:4 + nbl, :] = carry_ref[4:4 + nbl, :] + segsum

    x0 = x0c_ref[...].reshape(CHUNK, x0c_ref.shape[2]).astype(bf)
    dtraw0 = jax.lax.dot_general(
        x0, wdtp_ref[...], dimension_numbers=(((1,), (0,)), ((), ())),
        preferred_element_type=f32) + dtbp_ref[...]
    dt0 = jnp.maximum(dtraw0, 0.0) + jnp.log1p(jnp.exp(-jnp.abs(dtraw0)))
    a0 = (app_ref[...] * dt0).astype(bf)
    li = jax.lax.broadcasted_iota(jnp.int32, (CHUNK, CHUNK), 0)
    lj = jax.lax.broadcasted_iota(jnp.int32, (CHUNK, CHUNK), 1)
    ltri = lj <= li
    cum0 = jax.lax.dot_general(
        ltri.astype(bf), a0, dimension_numbers=(((1,), (0,)), ((), ())),
        preferred_element_type=f32)
    u = jnp.exp(cum0)
    v = jnp.exp(-cum0)
    ul = u[CHUNK - 1:CHUNK, :]
    for h in range(nh_loc):
        sl = slice(h * HP, (h + 1) * HP)
        ucol = u[:, h:h + 1]
        vcol = v[:, h:h + 1]
        ct = (conv[:, sl] * ucol).astype(bf)
        bv = (conv[:, part + h * HP:part + (h + 1) * HP] * vcol).astype(bf)
        xh = conv[:, 2 * part + h * HP:2 * part + (h + 1) * HP].astype(bf)
        g = jax.lax.dot_general(
            ct, bv, dimension_numbers=(((1,), (1,)), ((), ())),
            preferred_element_type=f32)
        gm = jnp.where(ltri, g, 0.0).astype(bf)
        yd = jax.lax.dot_general(
            gm, xh, dimension_numbers=(((1,), (0,)), ((), ())),
            preferred_element_type=f32)
        sh = state_ref[sl, :]
        yo = jax.lax.dot_general(
            ct, sh.astype(bf), dimension_numbers=(((1,), (0,)), ((), ())),
            preferred_element_type=f32)
        y_ref[:, sl] = yd + yo
        sc = jax.lax.dot_general(
            bv, xh, dimension_numbers=(((0,), (0,)), ((), ())),
            preferred_element_type=f32)
        state_ref[sl, :] = (sh + sc) * ul[0:1, h:h + 1]


def _ssd(proj0p, x, wdt_bf, wdtp_bf, cwp, cbp, ap_row, dtb_row,
         app_row, dtbp_row):
    nb, s, dm = x.shape
    dcc = proj0p.shape[1]
    nheads = ap_row.shape[1]
    nchunks = s // CHUNK
    slab = dcc // NCORES
    nbl = nb // NCORES
    kfn = functools.partial(_ssd_kernel, nheads)
    return pl.pallas_call(
        kfn,
        grid=(NCORES, nchunks),
        in_specs=[
            pl.BlockSpec((CHUNK, slab), lambda c, i: (i, c)),
            pl.BlockSpec((CHUNK, slab),
                         lambda c, i: (jnp.maximum(i - 1, 0), c)),
            pl.BlockSpec((nbl, CHUNK, dm), lambda c, i: (c, i, 0)),
            pl.BlockSpec((1, CHUNK, dm), lambda c, i: (0, i, 0)),
            pl.BlockSpec((dm, nheads), lambda c, i: (0, 0)),
            pl.BlockSpec((dm, 128), lambda c, i: (0, c)),
            pl.BlockSpec((D_CONV, slab), lambda c, i: (0, c)),
            pl.BlockSpec((1, slab), lambda c, i: (0, c)),
            pl.BlockSpec((1, nheads), lambda c, i: (0, 0)),
            pl.BlockSpec((1, nheads), lambda c, i: (0, 0)),
            pl.BlockSpec((1, 128), lambda c, i: (0, c)),
            pl.BlockSpec((1, 128), lambda c, i: (0, c)),
        ],
        out_specs=[
            pl.BlockSpec((CHUNK, slab // 3), lambda c, i: (i, c)),
            pl.BlockSpec((nbl, CHUNK, nheads), lambda c, i: (c, i, 0)),
        ],
        out_shape=[
            jax.ShapeDtypeStruct((s, dcc // 3), jnp.float32),
            jax.ShapeDtypeStruct((nb, s, nheads), jnp.float32),
        ],
        scratch_shapes=[
            pltpu.VMEM((slab // 3, HP), jnp.float32),
            pltpu.VMEM((8, nheads), jnp.float32),
        ],
        compiler_params=pltpu.CompilerParams(
            dimension_semantics=("core_parallel", "arbitrary")),
        name="conv_ssd_norm",
    )(proj0p, proj0p, x, x, wdt_bf, wdtp_bf, cwp, cbp, ap_row, dtb_row,
      app_row, dtbp_row)


def _out_kernel(nheads, y_ref, inv_ref, w_ref, o_ref):
    f32 = jnp.float32
    bm = y_ref.shape[0]
    di = y_ref.shape[1]
    hp = di // nheads
    inv = inv_ref[...].reshape(bm, nheads)
    invx = jnp.concatenate(
        [jnp.broadcast_to(inv[:, h:h + 1], (bm, hp)) for h in range(nheads)],
        axis=1)
    z = (y_ref[...] * invx).astype(jnp.bfloat16)
    o = jax.lax.dot_general(
        z, w_ref[...], dimension_numbers=(((1,), (0,)), ((), ())),
        preferred_element_type=f32)
    o_ref[...] = o.reshape(1, bm, o.shape[1])


def _out_proj(y0, invn, wot_bf):
    nb, s, nheads = invn.shape
    di = y0.shape[1]
    dm = wot_bf.shape[1]
    bm = 1024
    nbl = nb // NCORES
    kfn = functools.partial(_out_kernel, nheads)
    return pl.pallas_call(
        kfn,
        grid=(NCORES, s // bm, nbl),
        in_specs=[
            pl.BlockSpec((bm, di), lambda c, m, b: (m, 0)),
            pl.BlockSpec((1, bm, nheads),
                         lambda c, m, b: (c * nbl + b, m, 0)),
            pl.BlockSpec((di, dm), lambda c, m, b: (0, 0)),
        ],
        out_specs=pl.BlockSpec((1, bm, dm),
                               lambda c, m, b: (c * nbl + b, m, 0)),
        out_shape=jax.ShapeDtypeStruct((nb, s, dm), jnp.float32),
        compiler_params=pltpu.CompilerParams(
            dimension_semantics=("core_parallel", "parallel", "parallel")),
        name="scale_outproj",
    )(y0, invn, wot_bf)


def kernel(x, W_in, conv_w, conv_b, A_param, dt_bias, W_out):
    nb, s, dm = x.shape
    nheads = A_param.shape[0]
    dcc = conv_w.shape[0]
    nh_loc = nheads // NCORES

    def permute_cols(a):
        lead = a.shape[:-1]
        ap = a.reshape(*lead, 3, NCORES, nh_loc, HP)
        ap = jnp.moveaxis(ap, -4, -3)
        return ap.reshape(*lead, dcc)

    x0 = x[0]
    w1t_bf = permute_cols(W_in[:dcc].T).astype(jnp.bfloat16)
    wdt = W_in[dcc:].T
    wdt_bf = wdt.astype(jnp.bfloat16)
    wdtp = jnp.zeros((dm, NCORES * 128), jnp.float32)
    app_row = jnp.zeros((1, NCORES * 128), jnp.float32)
    dtbp_row = jnp.zeros((1, NCORES * 128), jnp.float32)
    for c in range(NCORES):
        hs = slice(c * nh_loc, (c + 1) * nh_loc)
        cs = slice(c * 128, c * 128 + nh_loc)
        wdtp = wdtp.at[:, cs].set(wdt[:, hs])
        app_row = app_row.at[0, cs].set(A_param[hs])
        dtbp_row = dtbp_row.at[0, cs].set(dt_bias[hs])
    wdtp_bf = wdtp.astype(jnp.bfloat16)
    cwp = permute_cols(conv_w.T)
    cbp = permute_cols(conv_b.reshape(1, dcc))
    ap_row = A_param.reshape(1, nheads)
    dtb_row = dt_bias.reshape(1, nheads)
    wot_bf = W_out.T.astype(jnp.bfloat16)

    proj0p = _proj_xbc(x0, w1t_bf)
    y0p, invn = _ssd(proj0p, x, wdt_bf, wdtp_bf, cwp, cbp, ap_row, dtb_row,
                     app_row, dtbp_row)
    return _out_proj(y0p, invn, wot_bf)
